# baseline (speedup 1.0000x reference)
import functools

import jax
import jax.numpy as jnp
from jax import lax
from jax.experimental import pallas as pl
from jax.experimental.pallas import tpu as pltpu
from jax.experimental.pallas import tpu_sc as plsc

N_NODES = 10000
N_EDGES = 320000
D = 128

N_PHASES = 2
E_PHASE = N_EDGES // N_PHASES

EDGE_BLOCK = 2000
NODE_BLOCK = 2000

NUM_SC = 2
NUM_SUB = 16
NW = NUM_SC * NUM_SUB
E_PER_W = E_PHASE // NW
CHUNK = 40
N_CHUNKS = E_PER_W // CHUNK
N_PER_SUB = 624
N_LAST_SUB = N_NODES - (NUM_SUB - 1) * N_PER_SUB

_sc_mesh = plsc.VectorSubcoreMesh(core_axis_name="c", subcore_axis_name="s")


def _gather_body(node_hbm, src_hbm, dst_hbm, gs_hbm, gd_hbm,
                 idx_s, idx_d, rows_s, rows_d, sem_s, sem_d):
    wid = lax.axis_index("c") * NUM_SUB + lax.axis_index("s")
    base0 = wid * E_PER_W

    @pl.loop(0, N_CHUNKS)
    def _(k):
        base = base0 + k * CHUNK
        pltpu.sync_copy(src_hbm.at[pl.ds(base, CHUNK)], idx_s)
        pltpu.sync_copy(dst_hbm.at[pl.ds(base, CHUNK)], idx_d)
        cp_s = pltpu.async_copy(node_hbm.at[idx_s], rows_s, sem_s)
        cp_d = pltpu.async_copy(node_hbm.at[idx_d], rows_d, sem_d)
        cp_s.wait()
        pltpu.sync_copy(rows_s, gs_hbm.at[pl.ds(base, CHUNK)])
        cp_d.wait()
        pltpu.sync_copy(rows_d, gd_hbm.at[pl.ds(base, CHUNK)])


def _sc_gather(node_rep, src, dst):
    fn = pl.kernel(
        _gather_body,
        out_type=(jax.ShapeDtypeStruct((E_PHASE, D), jnp.float32),
                  jax.ShapeDtypeStruct((E_PHASE, D), jnp.float32)),
        mesh=_sc_mesh,
        scratch_types=[
            pltpu.VMEM((CHUNK,), jnp.int32),
            pltpu.VMEM((CHUNK,), jnp.int32),
            pltpu.VMEM((CHUNK, D), jnp.float32),
            pltpu.VMEM((CHUNK, D), jnp.float32),
            pltpu.SemaphoreType.DMA,
            pltpu.SemaphoreType.DMA,
        ],
    )
    return fn(node_rep, src, dst)


def _scatter_body(edge_out_hbm, src_hbm, dst_hbm, zeros_hbm, part_hbm,
                  idx_s, idx_d, rows, acc, sem):
    c = lax.axis_index("c")
    s = lax.axis_index("s")
    wid = c * NUM_SUB + s

    @pl.when(s < NUM_SUB - 1)
    def _():
        pltpu.sync_copy(zeros_hbm.at[pl.ds(s * N_PER_SUB, N_PER_SUB)],
                        acc.at[pl.ds(s * N_PER_SUB, N_PER_SUB)])

    @pl.when(s == NUM_SUB - 1)
    def _():
        pltpu.sync_copy(zeros_hbm.at[pl.ds(s * N_PER_SUB, N_LAST_SUB)],
                        acc.at[pl.ds(s * N_PER_SUB, N_LAST_SUB)])

    plsc.subcore_barrier()

    base0 = wid * E_PER_W

    @pl.loop(0, N_CHUNKS)
    def _(k):
        base = base0 + k * CHUNK
        pltpu.sync_copy(src_hbm.at[pl.ds(base, CHUNK)], idx_s)
        pltpu.sync_copy(dst_hbm.at[pl.ds(base, CHUNK)], idx_d)
        pltpu.async_copy(edge_out_hbm.at[pl.ds(base, CHUNK)], rows, sem).wait()
        pltpu.sync_copy(rows, acc.at[idx_s], add=True)
        pltpu.sync_copy(rows, acc.at[idx_d], add=True)

    plsc.subcore_barrier()

    @pl.when(s < NUM_SUB - 1)
    def _():
        pltpu.sync_copy(acc.at[pl.ds(s * N_PER_SUB, N_PER_SUB)],
                        part_hbm.at[c].at[pl.ds(s * N_PER_SUB, N_PER_SUB)])

    @pl.when(s == NUM_SUB - 1)
    def _():
        pltpu.sync_copy(acc.at[pl.ds(s * N_PER_SUB, N_LAST_SUB)],
                        part_hbm.at[c].at[pl.ds(s * N_PER_SUB, N_LAST_SUB)])


def _sc_scatter(edge_out_slice, src, dst, zeros):
    fn = pl.kernel(
        _scatter_body,
        out_type=jax.ShapeDtypeStruct((NUM_SC, N_NODES, D), jnp.float32),
        mesh=_sc_mesh,
        scratch_types=[
            pltpu.VMEM((CHUNK,), jnp.int32),
            pltpu.VMEM((CHUNK,), jnp.int32),
            pltpu.VMEM((CHUNK, D), jnp.float32),
            pltpu.VMEM_SHARED((N_NODES, D), jnp.float32),
            pltpu.SemaphoreType.DMA,
        ],
    )
    return fn(edge_out_slice, src, dst, zeros)


def _make_mlp_body(n_extra):
    def body(*refs):
        a_ref = refs[0]
        extras = refs[1:1 + n_extra]
        w1_ref, b1_ref, w2_ref, b2_ref, out_ref = refs[1 + n_extra:]
        extra = extras[0][...]
        for e in extras[1:]:
            extra = extra + e[...]
        x = jnp.concatenate([a_ref[...], extra], axis=-1).astype(jnp.bfloat16)
        h = jnp.dot(x, w1_ref[...].astype(jnp.bfloat16),
                    preferred_element_type=jnp.float32)
        h = jnp.maximum(h + b1_ref[...], 0.0).astype(jnp.bfloat16)
        o = jnp.dot(h, w2_ref[...].astype(jnp.bfloat16),
                    preferred_element_type=jnp.float32)
        out_ref[...] = jnp.maximum(o + b2_ref[...], 0.0)
    return body


def _edge_mlp_phase(er_p, gs_p, gd_p, W1, bias1, W2, bias2):
    nblk = E_PHASE // EDGE_BLOCK
    row = lambda i: (i, 0)
    full = lambda i: (0, 0)
    return pl.pallas_call(
        _make_mlp_body(2),
        grid=(nblk,),
        in_specs=[
            pl.BlockSpec((EDGE_BLOCK, D), row),
            pl.BlockSpec((EDGE_BLOCK, D), row),
            pl.BlockSpec((EDGE_BLOCK, D), row),
            pl.BlockSpec((2 * D, 2 * D), full),
            pl.BlockSpec((1, 2 * D), full),
            pl.BlockSpec((2 * D, D), full),
            pl.BlockSpec((1, D), full),
        ],
        out_specs=pl.BlockSpec((EDGE_BLOCK, D), row),
        out_shape=jax.ShapeDtypeStruct((E_PHASE, D), jnp.float32),
    )(er_p, gs_p, gd_p, W1, bias1.reshape(1, -1), W2, bias2.reshape(1, -1))


def _node_mlp(node_rep, partials, W1, bias1, W2, bias2):
    n_extra = len(partials)
    row = lambda i: (i, 0)
    full = lambda i: (0, 0)
    return pl.pallas_call(
        _make_mlp_body(n_extra),
        grid=(N_NODES // NODE_BLOCK,),
        in_specs=[pl.BlockSpec((NODE_BLOCK, D), row)] * (1 + n_extra) + [
            pl.BlockSpec((2 * D, 2 * D), full),
            pl.BlockSpec((1, 2 * D), full),
            pl.BlockSpec((2 * D, D), full),
            pl.BlockSpec((1, D), full),
        ],
        out_specs=pl.BlockSpec((NODE_BLOCK, D), row),
        out_shape=jax.ShapeDtypeStruct((N_NODES, D), jnp.float32),
    )(node_rep, *partials, W1, bias1.reshape(1, -1), W2, bias2.reshape(1, -1))


def kernel(node_rep, edge_rep, edge_index, We1, be1, We2, be2, Wn1, bn1, Wn2, bn2):
    src = edge_index[0]
    dst = edge_index[1]
    zeros = jnp.zeros((N_NODES, D), jnp.float32)

    srcs = [src[p * E_PHASE:(p + 1) * E_PHASE] for p in range(N_PHASES)]
    dsts = [dst[p * E_PHASE:(p + 1) * E_PHASE] for p in range(N_PHASES)]
    ers = [edge_rep[p * E_PHASE:(p + 1) * E_PHASE] for p in range(N_PHASES)]

    gathered = [_sc_gather(node_rep, srcs[p], dsts[p]) for p in range(N_PHASES)]

    eo_slices = []
    for p in range(N_PHASES):
        gs_p, gd_p = gathered[p]
        eo_slices.append(_edge_mlp_phase(ers[p], gs_p, gd_p, We1, be1, We2, be2))

    partials = []
    for p in range(N_PHASES):
        part = _sc_scatter(eo_slices[p], srcs[p], dsts[p], zeros)
        partials.extend([part[0], part[1]])

    edge_out = jnp.concatenate(eo_slices, axis=0)
    node_out = _node_mlp(node_rep, partials, Wn1, bn1, Wn2, bn2)
    return (node_out, edge_out)


# speedup vs baseline: 1.8807x; 1.8807x over previous
import functools

import jax
import jax.numpy as jnp
from jax import lax
from jax.experimental import pallas as pl
from jax.experimental.pallas import tpu as pltpu
from jax.experimental.pallas import tpu_sc as plsc

N_NODES = 10000
N_EDGES = 320000
D = 128

NUM_SC = 2
NUM_SUB = 16
NW = NUM_SC * NUM_SUB
CHUNK = 80
NB = 3

NC_PHASES = (63, 62)
E_PHASES = tuple(NW * CHUNK * nc for nc in NC_PHASES)
N_PHASES = len(NC_PHASES)
assert sum(E_PHASES) == N_EDGES

EDGE_BLOCK = 2560
NODE_BLOCK = 2000

N_PER_SUB = 624
N_LAST_SUB = N_NODES - (NUM_SUB - 1) * N_PER_SUB

_sc_mesh = plsc.VectorSubcoreMesh(core_axis_name="c", subcore_axis_name="s")


def _make_gather_body(nc):
    def body(node_hbm, src_hbm, dst_hbm, gs_hbm, gd_hbm,
             idxs, idxd, rows_s, rows_d,
             isem_s, isem_d, gsem_s, gsem_d, ssem_s, ssem_d):
        wid = lax.axis_index("c") * NUM_SUB + lax.axis_index("s")
        base0 = wid * nc * CHUNK

        def fire_idx(t, b):
            base = base0 + t * CHUNK
            pltpu.async_copy(src_hbm.at[pl.ds(base, CHUNK)], idxs.at[b],
                             isem_s.at[b])
            pltpu.async_copy(dst_hbm.at[pl.ds(base, CHUNK)], idxd.at[b],
                             isem_d.at[b])

        def wait_idx(b):
            pltpu.make_async_copy(src_hbm.at[pl.ds(base0, CHUNK)], idxs.at[b],
                                  isem_s.at[b]).wait()
            pltpu.make_async_copy(dst_hbm.at[pl.ds(base0, CHUNK)], idxd.at[b],
                                  isem_d.at[b]).wait()

        def fire_gather(b):
            pltpu.async_copy(node_hbm.at[idxs.at[b]], rows_s.at[b],
                             gsem_s.at[b])
            pltpu.async_copy(node_hbm.at[idxd.at[b]], rows_d.at[b],
                             gsem_d.at[b])

        def wait_gather(b):
            pltpu.make_async_copy(node_hbm.at[idxs.at[b]], rows_s.at[b],
                                  gsem_s.at[b]).wait()
            pltpu.make_async_copy(node_hbm.at[idxd.at[b]], rows_d.at[b],
                                  gsem_d.at[b]).wait()

        def store(t, b):
            base = base0 + t * CHUNK
            pltpu.async_copy(rows_s.at[b], gs_hbm.at[pl.ds(base, CHUNK)],
                             ssem_s.at[b])
            pltpu.async_copy(rows_d.at[b], gd_hbm.at[pl.ds(base, CHUNK)],
                             ssem_d.at[b])

        def wait_store(b):
            pltpu.make_async_copy(rows_s.at[b], gs_hbm.at[pl.ds(base0, CHUNK)],
                                  ssem_s.at[b]).wait()
            pltpu.make_async_copy(rows_d.at[b], gd_hbm.at[pl.ds(base0, CHUNK)],
                                  ssem_d.at[b]).wait()

        @pl.loop(0, nc + 2)
        def _(t):
            @pl.when(t < nc)
            def _():
                fire_idx(t, lax.rem(t, NB))

            g = t - 1

            @pl.when(jnp.logical_and(g >= 0, g < nc))
            def _():
                bg = lax.rem(g, NB)
                wait_idx(bg)

                @pl.when(g >= NB)
                def _():
                    wait_store(bg)

                fire_gather(bg)

            c = t - 2

            @pl.when(c >= 0)
            def _():
                bc = lax.rem(c, NB)
                wait_gather(bc)
                store(c, bc)

        for b in range(NB):
            wait_store(b)

    return body


def _sc_gather(node_rep, src, dst, nc):
    e_phase = NW * CHUNK * nc
    fn = pl.kernel(
        _make_gather_body(nc),
        out_type=(jax.ShapeDtypeStruct((e_phase, D), jnp.float32),
                  jax.ShapeDtypeStruct((e_phase, D), jnp.float32)),
        mesh=_sc_mesh,
        scratch_types=[
            pltpu.VMEM((NB, CHUNK), jnp.int32),
            pltpu.VMEM((NB, CHUNK), jnp.int32),
            pltpu.VMEM((NB, CHUNK, D), jnp.float32),
            pltpu.VMEM((NB, CHUNK, D), jnp.float32),
            pltpu.SemaphoreType.DMA((NB,)),
            pltpu.SemaphoreType.DMA((NB,)),
            pltpu.SemaphoreType.DMA((NB,)),
            pltpu.SemaphoreType.DMA((NB,)),
            pltpu.SemaphoreType.DMA((NB,)),
            pltpu.SemaphoreType.DMA((NB,)),
        ],
    )
    return fn(node_rep, src, dst)


def _make_scatter_body(nc):
    def body(eo_hbm, src_hbm, dst_hbm, zeros_hbm, part_hbm,
             idxs, idxd, rows, acc, isem_s, isem_d, gsem):
        c = lax.axis_index("c")
        s = lax.axis_index("s")
        wid = c * NUM_SUB + s
        base0 = wid * nc * CHUNK

        @pl.when(s < NUM_SUB - 1)
        def _():
            pltpu.sync_copy(zeros_hbm.at[pl.ds(s * N_PER_SUB, N_PER_SUB)],
                            acc.at[pl.ds(s * N_PER_SUB, N_PER_SUB)])

        @pl.when(s == NUM_SUB - 1)
        def _():
            pltpu.sync_copy(zeros_hbm.at[pl.ds(s * N_PER_SUB, N_LAST_SUB)],
                            acc.at[pl.ds(s * N_PER_SUB, N_LAST_SUB)])

        plsc.subcore_barrier()

        def fire(t, b):
            base = base0 + t * CHUNK
            pltpu.async_copy(src_hbm.at[pl.ds(base, CHUNK)], idxs.at[b],
                             isem_s.at[b])
            pltpu.async_copy(dst_hbm.at[pl.ds(base, CHUNK)], idxd.at[b],
                             isem_d.at[b])
            pltpu.async_copy(eo_hbm.at[pl.ds(base, CHUNK)], rows.at[b],
                             gsem.at[b])

        def wait_chunk(b):
            pltpu.make_async_copy(src_hbm.at[pl.ds(base0, CHUNK)], idxs.at[b],
                                  isem_s.at[b]).wait()
            pltpu.make_async_copy(dst_hbm.at[pl.ds(base0, CHUNK)], idxd.at[b],
                                  isem_d.at[b]).wait()
            pltpu.make_async_copy(eo_hbm.at[pl.ds(base0, CHUNK)], rows.at[b],
                                  gsem.at[b]).wait()

        @pl.loop(0, nc + NB - 1)
        def _(t):
            @pl.when(t < nc)
            def _():
                fire(t, lax.rem(t, NB))

            comp = t - (NB - 1)

            @pl.when(comp >= 0)
            def _():
                bc = lax.rem(comp, NB)
                wait_chunk(bc)
                pltpu.sync_copy(rows.at[bc], acc.at[idxs.at[bc]], add=True)
                pltpu.sync_copy(rows.at[bc], acc.at[idxd.at[bc]], add=True)

        plsc.subcore_barrier()

        @pl.when(s < NUM_SUB - 1)
        def _():
            pltpu.sync_copy(acc.at[pl.ds(s * N_PER_SUB, N_PER_SUB)],
                            part_hbm.at[c].at[pl.ds(s * N_PER_SUB, N_PER_SUB)])

        @pl.when(s == NUM_SUB - 1)
        def _():
            pltpu.sync_copy(acc.at[pl.ds(s * N_PER_SUB, N_LAST_SUB)],
                            part_hbm.at[c].at[pl.ds(s * N_PER_SUB, N_LAST_SUB)])

    return body


def _sc_scatter(edge_out_slice, src, dst, zeros, nc):
    fn = pl.kernel(
        _make_scatter_body(nc),
        out_type=jax.ShapeDtypeStruct((NUM_SC, N_NODES, D), jnp.float32),
        mesh=_sc_mesh,
        scratch_types=[
            pltpu.VMEM((NB, CHUNK), jnp.int32),
            pltpu.VMEM((NB, CHUNK), jnp.int32),
            pltpu.VMEM((NB, CHUNK, D), jnp.float32),
            pltpu.VMEM_SHARED((N_NODES, D), jnp.float32),
            pltpu.SemaphoreType.DMA((NB,)),
            pltpu.SemaphoreType.DMA((NB,)),
            pltpu.SemaphoreType.DMA((NB,)),
        ],
    )
    return fn(edge_out_slice, src, dst, zeros)


def _make_mlp_body(n_extra):
    def body(*refs):
        a_ref = refs[0]
        extras = refs[1:1 + n_extra]
        w1_ref, b1_ref, w2_ref, b2_ref, out_ref = refs[1 + n_extra:]
        extra = extras[0][...]
        for e in extras[1:]:
            extra = extra + e[...]
        x = jnp.concatenate([a_ref[...], extra], axis=-1).astype(jnp.bfloat16)
        h = jnp.dot(x, w1_ref[...].astype(jnp.bfloat16),
                    preferred_element_type=jnp.float32)
        h = jnp.maximum(h + b1_ref[...], 0.0).astype(jnp.bfloat16)
        o = jnp.dot(h, w2_ref[...].astype(jnp.bfloat16),
                    preferred_element_type=jnp.float32)
        out_ref[...] = jnp.maximum(o + b2_ref[...], 0.0)
    return body


def _edge_mlp_phase(er_p, gs_p, gd_p, W1, bias1, W2, bias2):
    n = er_p.shape[0]
    row = lambda i: (i, 0)
    full = lambda i: (0, 0)
    return pl.pallas_call(
        _make_mlp_body(2),
        grid=(n // EDGE_BLOCK,),
        in_specs=[
            pl.BlockSpec((EDGE_BLOCK, D), row),
            pl.BlockSpec((EDGE_BLOCK, D), row),
            pl.BlockSpec((EDGE_BLOCK, D), row),
            pl.BlockSpec((2 * D, 2 * D), full),
            pl.BlockSpec((1, 2 * D), full),
            pl.BlockSpec((2 * D, D), full),
            pl.BlockSpec((1, D), full),
        ],
        out_specs=pl.BlockSpec((EDGE_BLOCK, D), row),
        out_shape=jax.ShapeDtypeStruct((n, D), jnp.float32),
    )(er_p, gs_p, gd_p, W1, bias1.reshape(1, -1), W2, bias2.reshape(1, -1))


def _node_mlp(node_rep, partials, W1, bias1, W2, bias2):
    n_extra = len(partials)
    row = lambda i: (i, 0)
    full = lambda i: (0, 0)
    return pl.pallas_call(
        _make_mlp_body(n_extra),
        grid=(N_NODES // NODE_BLOCK,),
        in_specs=[pl.BlockSpec((NODE_BLOCK, D), row)] * (1 + n_extra) + [
            pl.BlockSpec((2 * D, 2 * D), full),
            pl.BlockSpec((1, 2 * D), full),
            pl.BlockSpec((2 * D, D), full),
            pl.BlockSpec((1, D), full),
        ],
        out_specs=pl.BlockSpec((NODE_BLOCK, D), row),
        out_shape=jax.ShapeDtypeStruct((N_NODES, D), jnp.float32),
    )(node_rep, *partials, W1, bias1.reshape(1, -1), W2, bias2.reshape(1, -1))


def kernel(node_rep, edge_rep, edge_index, We1, be1, We2, be2, Wn1, bn1, Wn2, bn2):
    src = edge_index[0]
    dst = edge_index[1]
    zeros = jnp.zeros((N_NODES, D), jnp.float32)

    bounds = [0]
    for e in E_PHASES:
        bounds.append(bounds[-1] + e)

    srcs, dsts, ers = [], [], []
    for p in range(N_PHASES):
        lo, hi = bounds[p], bounds[p + 1]
        srcs.append(src[lo:hi])
        dsts.append(dst[lo:hi])
        ers.append(edge_rep[lo:hi])

    gathered = [_sc_gather(node_rep, srcs[p], dsts[p], NC_PHASES[p])
                for p in range(N_PHASES)]

    eo_slices = []
    for p in range(N_PHASES):
        gs_p, gd_p = gathered[p]
        eo_slices.append(_edge_mlp_phase(ers[p], gs_p, gd_p, We1, be1, We2, be2))

    partials = []
    for p in range(N_PHASES):
        part = _sc_scatter(eo_slices[p], srcs[p], dsts[p], zeros, NC_PHASES[p])
        partials.extend([part[0], part[1]])

    edge_out = jnp.concatenate(eo_slices, axis=0)
    node_out = _node_mlp(node_rep, partials, Wn1, bn1, Wn2, bn2)
    return (node_out, edge_out)
